# Initial kernel scaffold; baseline (speedup 1.0000x reference)
import functools

import jax
import jax.numpy as jnp
from jax import lax
from jax.experimental import pallas as pl
from jax.experimental.pallas import tpu as pltpu
from jax.experimental.pallas import tpu_sc as plsc

N = 10000
NP = 10240
D = 128
E = 320000
NW = 32
CPW = 79
CH = 128
EP = NW * CPW * CH
RPT = NP // 16
BLK = 1280

f32 = jnp.float32

_mesh = plsc.VectorSubcoreMesh(core_axis_name="c", subcore_axis_name="s")


@functools.partial(
    pl.kernel,
    mesh=_mesh,
    out_type=jax.ShapeDtypeStruct((2 * NP, 8), f32),
    scratch_types=[
        pltpu.VMEM_SHARED((NP, 8), f32),
        pltpu.VMEM((CPW, CH), jnp.int32),
        pltpu.VMEM((CH, 8), f32),
    ],
)
def _deg_kernel(dst_hbm, ones_hbm, zeros_hbm, out_hbm, acc, idx, ones_v):
    c = lax.axis_index("c")
    s = lax.axis_index("s")
    w = c * 16 + s
    pltpu.sync_copy(zeros_hbm, acc.at[pl.ds(s * RPT, RPT)])
    pltpu.sync_copy(ones_hbm, ones_v)
    pltpu.sync_copy(dst_hbm.at[pl.ds(w * CPW, CPW)], idx)
    plsc.subcore_barrier()

    def body(j, carry):
        pltpu.sync_copy(ones_v, acc.at[idx.at[j]], add=True)
        return carry

    lax.fori_loop(0, CPW, body, 0)
    plsc.subcore_barrier()
    pltpu.sync_copy(acc.at[pl.ds(s * RPT, RPT)],
                    out_hbm.at[pl.ds(c * NP + s * RPT, RPT)])


@functools.partial(
    pl.kernel,
    mesh=_mesh,
    out_type=jax.ShapeDtypeStruct((2 * NP, D), f32),
    scratch_types=[
        pltpu.VMEM_SHARED((NP, D), f32),
        pltpu.VMEM((CPW, CH), jnp.int32),
        pltpu.VMEM((CPW, CH), jnp.int32),
        pltpu.VMEM((CH, D), f32),
        pltpu.SemaphoreType.DMA,
    ],
)
def _agg_kernel(g_hbm, src_hbm, dst_hbm, zeros_hbm, out_hbm,
                acc, idx_s, idx_d, rows, sem):
    c = lax.axis_index("c")
    s = lax.axis_index("s")
    w = c * 16 + s
    pltpu.sync_copy(zeros_hbm, acc.at[pl.ds(s * RPT, RPT)])
    pltpu.sync_copy(src_hbm.at[pl.ds(w * CPW, CPW)], idx_s)
    pltpu.sync_copy(dst_hbm.at[pl.ds(w * CPW, CPW)], idx_d)
    plsc.subcore_barrier()

    def body(j, carry):
        pltpu.async_copy(g_hbm.at[idx_s.at[j]], rows, sem).wait()
        pltpu.sync_copy(rows, acc.at[idx_d.at[j]], add=True)
        return carry

    lax.fori_loop(0, CPW, body, 0)
    plsc.subcore_barrier()
    pltpu.sync_copy(acc.at[pl.ds(s * RPT, RPT)],
                    out_hbm.at[pl.ds(c * NP + s * RPT, RPT)])


def _dinv_block(i, da_ref, db_ref):
    row = i * BLK + lax.broadcasted_iota(jnp.int32, (BLK, 1), 0)
    deg = da_ref[:, 0:1] + db_ref[:, 0:1] + 1.0
    return jnp.where(row < N, lax.rsqrt(deg), 0.0)


def _tc_first_body(x_ref, w_ref, da_ref, db_ref, g_ref):
    dinv = _dinv_block(pl.program_id(0), da_ref, db_ref)
    g_ref[...] = dinv * jnp.dot(x_ref[...], w_ref[...],
                                preferred_element_type=f32)


def _tc_mid_body(sa_ref, sb_ref, gp_ref, da_ref, db_ref, b_ref, w_ref, g_ref):
    dinv = _dinv_block(pl.program_id(0), da_ref, db_ref)
    h = jnp.maximum(
        dinv * (sa_ref[...] + sb_ref[...] + gp_ref[...]) + b_ref[...], 0.0)
    g_ref[...] = dinv * jnp.dot(h, w_ref[...], preferred_element_type=f32)


def _tc_final_body(sa_ref, sb_ref, gp_ref, da_ref, db_ref, b_ref, o_ref):
    dinv = _dinv_block(pl.program_id(0), da_ref, db_ref)
    o_ref[...] = dinv * (sa_ref[...] + sb_ref[...] + gp_ref[...]) + b_ref[...]


_row_spec = pl.BlockSpec((BLK, D), lambda i: (i, 0))
_deg_spec = pl.BlockSpec((BLK, 8), lambda i: (i, 0))
_w_spec = pl.BlockSpec((D, D), lambda i: (0, 0))
_b_spec = pl.BlockSpec((1, D), lambda i: (0, 0))
_out_sds = jax.ShapeDtypeStruct((NP, D), f32)


def _tc_first(x, w, da, db):
    return pl.pallas_call(
        _tc_first_body, grid=(NP // BLK,),
        in_specs=[_row_spec, _w_spec, _deg_spec, _deg_spec],
        out_specs=_row_spec, out_shape=_out_sds)(x, w, da, db)


def _tc_mid(sa, sb, gp, da, db, b, w):
    return pl.pallas_call(
        _tc_mid_body, grid=(NP // BLK,),
        in_specs=[_row_spec, _row_spec, _row_spec, _deg_spec, _deg_spec,
                  _b_spec, _w_spec],
        out_specs=_row_spec, out_shape=_out_sds)(sa, sb, gp, da, db, b, w)


def _tc_final(sa, sb, gp, da, db, b):
    return pl.pallas_call(
        _tc_final_body, grid=(NP // BLK,),
        in_specs=[_row_spec, _row_spec, _row_spec, _deg_spec, _deg_spec,
                  _b_spec],
        out_specs=_row_spec, out_shape=_out_sds)(sa, sb, gp, da, db, b)


def kernel(x, edge_index, W0, b0, W1, b1, W2, b2):
    pad = jnp.full((EP - E,), N, jnp.int32)
    src = jnp.concatenate([edge_index[0], pad]).reshape(NW * CPW, CH)
    dst = jnp.concatenate([edge_index[1], pad]).reshape(NW * CPW, CH)
    x_pad = jnp.zeros((NP, D), f32).at[:N].set(x.astype(f32))
    ones8 = jnp.ones((CH, 8), f32)
    zeros8 = jnp.zeros((RPT, 8), f32)
    zerosD = jnp.zeros((RPT, D), f32)
    b0r, b1r, b2r = (b.reshape(1, D) for b in (b0, b1, b2))

    deg2 = _deg_kernel(dst, ones8, zeros8)
    da, db = deg2[:NP], deg2[NP:]

    g = _tc_first(x_pad, W0, da, db)
    s2 = _agg_kernel(g, src, dst, zerosD)
    g = _tc_mid(s2[:NP], s2[NP:], g, da, db, b0r, W1)
    s2 = _agg_kernel(g, src, dst, zerosD)
    g = _tc_mid(s2[:NP], s2[NP:], g, da, db, b1r, W2)
    s2 = _agg_kernel(g, src, dst, zerosD)
    out = _tc_final(s2[:NP], s2[NP:], g, da, db, b2r)
    return out[:N]


# speedup vs baseline: 6.9597x; 6.9597x over previous
import functools

import jax
import jax.numpy as jnp
from jax import lax
from jax.experimental import pallas as pl
from jax.experimental.pallas import tpu as pltpu
from jax.experimental.pallas import tpu_sc as plsc

N = 10000
NP = 10240
D = 128
E = 320000
NW = 32
CPW = 80
CH = 128
EP = NW * CPW * CH
RPT = NP // 16
BLK = 1280

f32 = jnp.float32

_mesh = plsc.VectorSubcoreMesh(core_axis_name="c", subcore_axis_name="s")


@functools.partial(
    pl.kernel,
    mesh=_mesh,
    out_type=jax.ShapeDtypeStruct((2 * NP, D), f32),
    scratch_types=[
        pltpu.VMEM_SHARED((NP, D), f32),
        pltpu.VMEM((CPW, CH), jnp.int32),
        pltpu.VMEM((CH, D), f32),
    ],
)
def _deg_kernel(dst_hbm, ones_hbm, zeros_hbm, out_hbm, acc, idx, ones_v):
    c = lax.axis_index("c")
    s = lax.axis_index("s")
    w = c * 16 + s
    pltpu.sync_copy(zeros_hbm, acc.at[pl.ds(s * RPT, RPT)])
    pltpu.sync_copy(ones_hbm, ones_v)
    pltpu.sync_copy(dst_hbm.at[pl.ds(w * CPW, CPW)], idx)
    plsc.subcore_barrier()

    def body(j, carry):
        pltpu.sync_copy(ones_v, acc.at[idx.at[j]], add=True)
        return carry

    lax.fori_loop(0, CPW, body, 0)
    plsc.subcore_barrier()
    pltpu.sync_copy(acc.at[pl.ds(s * RPT, RPT)],
                    out_hbm.at[pl.ds(c * NP + s * RPT, RPT)])


@functools.partial(
    pl.kernel,
    mesh=_mesh,
    out_type=jax.ShapeDtypeStruct((2 * NP, D), f32),
    scratch_types=[
        pltpu.VMEM_SHARED((NP, D), f32),
        pltpu.VMEM((CPW, CH), jnp.int32),
        pltpu.VMEM((CPW, CH), jnp.int32),
        pltpu.VMEM((CH, D), f32),
        pltpu.SemaphoreType.DMA,
    ],
)
def _agg_kernel(g_hbm, src_hbm, dst_hbm, zeros_hbm, out_hbm,
                acc, idx_s, idx_d, rows, sem):
    c = lax.axis_index("c")
    s = lax.axis_index("s")
    w = c * 16 + s
    pltpu.sync_copy(zeros_hbm, acc.at[pl.ds(s * RPT, RPT)])
    pltpu.sync_copy(src_hbm.at[pl.ds(w * CPW, CPW)], idx_s)
    pltpu.sync_copy(dst_hbm.at[pl.ds(w * CPW, CPW)], idx_d)
    plsc.subcore_barrier()

    def body(j, carry):
        pltpu.async_copy(g_hbm.at[idx_s.at[j]], rows, sem).wait()
        pltpu.sync_copy(rows, acc.at[idx_d.at[j]], add=True)
        return carry

    lax.fori_loop(0, CPW, body, 0)
    plsc.subcore_barrier()
    pltpu.sync_copy(acc.at[pl.ds(s * RPT, RPT)],
                    out_hbm.at[pl.ds(c * NP + s * RPT, RPT)])


def _dinv_block(i, da_ref, db_ref):
    row = i * BLK + lax.broadcasted_iota(jnp.int32, (BLK, 1), 0)
    deg = da_ref[:, 0:1] + db_ref[:, 0:1] + 1.0
    return jnp.where(row < N, lax.rsqrt(deg), 0.0)


def _tc_first_body(x_ref, w_ref, da_ref, db_ref, g_ref):
    dinv = _dinv_block(pl.program_id(0), da_ref, db_ref)
    g_ref[...] = dinv * jnp.dot(x_ref[...], w_ref[...],
                                preferred_element_type=f32)


def _tc_mid_body(sa_ref, sb_ref, gp_ref, da_ref, db_ref, b_ref, w_ref, g_ref):
    dinv = _dinv_block(pl.program_id(0), da_ref, db_ref)
    h = jnp.maximum(
        dinv * (sa_ref[...] + sb_ref[...] + gp_ref[...]) + b_ref[...], 0.0)
    g_ref[...] = dinv * jnp.dot(h, w_ref[...], preferred_element_type=f32)


def _tc_final_body(sa_ref, sb_ref, gp_ref, da_ref, db_ref, b_ref, o_ref):
    dinv = _dinv_block(pl.program_id(0), da_ref, db_ref)
    o_ref[...] = dinv * (sa_ref[...] + sb_ref[...] + gp_ref[...]) + b_ref[...]


_row_spec = pl.BlockSpec((BLK, D), lambda i: (i, 0))
_deg_spec = _row_spec
_w_spec = pl.BlockSpec((D, D), lambda i: (0, 0))
_b_spec = pl.BlockSpec((1, D), lambda i: (0, 0))
_out_sds = jax.ShapeDtypeStruct((NP, D), f32)


def _tc_first(x, w, da, db):
    return pl.pallas_call(
        _tc_first_body, grid=(NP // BLK,),
        in_specs=[_row_spec, _w_spec, _deg_spec, _deg_spec],
        out_specs=_row_spec, out_shape=_out_sds)(x, w, da, db)


def _tc_mid(sa, sb, gp, da, db, b, w):
    return pl.pallas_call(
        _tc_mid_body, grid=(NP // BLK,),
        in_specs=[_row_spec, _row_spec, _row_spec, _deg_spec, _deg_spec,
                  _b_spec, _w_spec],
        out_specs=_row_spec, out_shape=_out_sds)(sa, sb, gp, da, db, b, w)


def _tc_final(sa, sb, gp, da, db, b):
    return pl.pallas_call(
        _tc_final_body, grid=(NP // BLK,),
        in_specs=[_row_spec, _row_spec, _row_spec, _deg_spec, _deg_spec,
                  _b_spec],
        out_specs=_row_spec, out_shape=_out_sds)(sa, sb, gp, da, db, b)


def kernel(x, edge_index, W0, b0, W1, b1, W2, b2):
    pad = jnp.full((EP - E,), N, jnp.int32)
    src = jnp.concatenate([edge_index[0], pad]).reshape(NW * CPW, CH)
    dst = jnp.concatenate([edge_index[1], pad]).reshape(NW * CPW, CH)
    x_pad = jnp.zeros((NP, D), f32).at[:N].set(x.astype(f32))
    onesD = jnp.ones((CH, D), f32)
    zerosD = jnp.zeros((RPT, D), f32)
    b0r, b1r, b2r = (b.reshape(1, D) for b in (b0, b1, b2))

    deg2 = _deg_kernel(dst, onesD, zerosD)
    da, db = deg2[:NP], deg2[NP:]

    g = _tc_first(x_pad, W0, da, db)
    s2 = _agg_kernel(g, src, dst, zerosD)
    g = _tc_mid(s2[:NP], s2[NP:], g, da, db, b0r, W1)
    s2 = _agg_kernel(g, src, dst, zerosD)
    g = _tc_mid(s2[:NP], s2[NP:], g, da, db, b1r, W2)
    s2 = _agg_kernel(g, src, dst, zerosD)
    out = _tc_final(s2[:NP], s2[NP:], g, da, db, b2r)
    return out[:N]


# speedup vs baseline: 7.7537x; 1.1141x over previous
import functools

import jax
import jax.numpy as jnp
from jax import lax
from jax.experimental import pallas as pl
from jax.experimental.pallas import tpu as pltpu
from jax.experimental.pallas import tpu_sc as plsc

N = 10000
NP = 10240
D = 128
E = 320000
NW = 32
CPW = 80
CH = 128
EP = NW * CPW * CH
RPT = NP // 16
IBLK = 16
BLK = 1280

f32 = jnp.float32

_mesh = plsc.VectorSubcoreMesh(core_axis_name="c", subcore_axis_name="s")


@functools.partial(
    pl.kernel,
    mesh=_mesh,
    out_type=jax.ShapeDtypeStruct((2 * NP, D), f32),
    scratch_types=[
        pltpu.VMEM_SHARED((NP, D), f32),
        pltpu.VMEM((CPW, CH), jnp.int32),
        pltpu.VMEM((CH, D), f32),
    ],
)
def _deg_kernel(dst_hbm, ones_hbm, zeros_hbm, out_hbm, acc, idx, ones_v):
    c = lax.axis_index("c")
    s = lax.axis_index("s")
    w = c * 16 + s
    pltpu.sync_copy(zeros_hbm, acc.at[pl.ds(s * RPT, RPT)])
    pltpu.sync_copy(ones_hbm, ones_v)
    pltpu.sync_copy(dst_hbm.at[pl.ds(w * CPW, CPW)], idx)
    plsc.subcore_barrier()

    def body(j, carry):
        pltpu.sync_copy(ones_v, acc.at[idx.at[j]], add=True)
        return carry

    lax.fori_loop(0, CPW, body, 0)
    plsc.subcore_barrier()
    pltpu.sync_copy(acc.at[pl.ds(s * RPT, RPT)],
                    out_hbm.at[pl.ds(c * NP + s * RPT, RPT)])


@functools.partial(
    pl.kernel,
    mesh=_mesh,
    out_type=jax.ShapeDtypeStruct((2 * NP, D), f32),
    scratch_types=[
        pltpu.VMEM_SHARED((NP, D), f32),
        pltpu.VMEM((3, IBLK, CH), jnp.int32),
        pltpu.VMEM((3, IBLK, CH), jnp.int32),
        pltpu.VMEM((CH, D), f32),
        pltpu.VMEM((CH, D), f32),
        pltpu.SemaphoreType.DMA,
        pltpu.SemaphoreType.DMA,
        pltpu.SemaphoreType.DMA,
    ],
)
def _agg_kernel(g_hbm, src_hbm, dst_hbm, zeros_hbm, out_hbm,
                acc, idx_s, idx_d, rows0, rows1, sem0, sem1, semi):
    c = lax.axis_index("c")
    s = lax.axis_index("s")
    w = c * 16 + s
    base = w * CPW
    pltpu.sync_copy(zeros_hbm, acc.at[pl.ds(s * RPT, RPT)])

    def iload(b, slot):
        pltpu.make_async_copy(
            src_hbm.at[pl.ds(base + b * IBLK, IBLK)], idx_s.at[slot], semi).start()
        pltpu.make_async_copy(
            dst_hbm.at[pl.ds(base + b * IBLK, IBLK)], idx_d.at[slot], semi).start()

    def iwait(b, slot):
        pltpu.make_async_copy(
            src_hbm.at[pl.ds(base + b * IBLK, IBLK)], idx_s.at[slot], semi).wait()
        pltpu.make_async_copy(
            dst_hbm.at[pl.ds(base + b * IBLK, IBLK)], idx_d.at[slot], semi).wait()

    def gather(j, buf, sem):
        return pltpu.make_async_copy(
            g_hbm.at[idx_s.at[(j // IBLK) % 3, j % IBLK]], buf, sem)

    def scat(j, buf):
        pltpu.sync_copy(buf, acc.at[idx_d.at[(j // IBLK) % 3, j % IBLK]],
                        add=True)

    iload(0, 0)
    iwait(0, 0)
    iload(1, 1)
    plsc.subcore_barrier()

    gather(0, rows0, sem0).start()

    def body(i, carry):
        j = 2 * i
        gather(j + 1, rows1, sem1).start()
        gather(j, rows0, sem0).wait()
        scat(j, rows0)

        @pl.when(i < CPW // 2 - 1)
        def _():
            @pl.when((j + 2) % IBLK == 0)
            def _():
                b2 = (j + 2) // IBLK
                iwait(b2, b2 % 3)

                @pl.when(b2 + 1 < CPW // IBLK)
                def _():
                    iload(b2 + 1, (b2 + 1) % 3)

            gather(j + 2, rows0, sem0).start()

        gather(j + 1, rows1, sem1).wait()
        scat(j + 1, rows1)
        return carry

    lax.fori_loop(0, CPW // 2, body, 0)
    plsc.subcore_barrier()
    pltpu.sync_copy(acc.at[pl.ds(s * RPT, RPT)],
                    out_hbm.at[pl.ds(c * NP + s * RPT, RPT)])


def _dinv_block(i, da_ref, db_ref):
    row = i * BLK + lax.broadcasted_iota(jnp.int32, (BLK, 1), 0)
    deg = da_ref[:, 0:1] + db_ref[:, 0:1] + 1.0
    return jnp.where(row < N, lax.rsqrt(deg), 0.0)


def _tc_first_body(x_ref, w_ref, da_ref, db_ref, g_ref):
    dinv = _dinv_block(pl.program_id(0), da_ref, db_ref)
    g_ref[...] = dinv * jnp.dot(x_ref[...], w_ref[...],
                                preferred_element_type=f32)


def _tc_mid_body(sa_ref, sb_ref, gp_ref, da_ref, db_ref, b_ref, w_ref, g_ref):
    dinv = _dinv_block(pl.program_id(0), da_ref, db_ref)
    h = jnp.maximum(
        dinv * (sa_ref[...] + sb_ref[...] + gp_ref[...]) + b_ref[...], 0.0)
    g_ref[...] = dinv * jnp.dot(h, w_ref[...], preferred_element_type=f32)


def _tc_final_body(sa_ref, sb_ref, gp_ref, da_ref, db_ref, b_ref, o_ref):
    dinv = _dinv_block(pl.program_id(0), da_ref, db_ref)
    o_ref[...] = dinv * (sa_ref[...] + sb_ref[...] + gp_ref[...]) + b_ref[...]


_row_spec = pl.BlockSpec((BLK, D), lambda i: (i, 0))
_deg_spec = _row_spec
_w_spec = pl.BlockSpec((D, D), lambda i: (0, 0))
_b_spec = pl.BlockSpec((1, D), lambda i: (0, 0))
_out_sds = jax.ShapeDtypeStruct((NP, D), f32)


def _tc_first(x, w, da, db):
    return pl.pallas_call(
        _tc_first_body, grid=(NP // BLK,),
        in_specs=[_row_spec, _w_spec, _deg_spec, _deg_spec],
        out_specs=_row_spec, out_shape=_out_sds)(x, w, da, db)


def _tc_mid(sa, sb, gp, da, db, b, w):
    return pl.pallas_call(
        _tc_mid_body, grid=(NP // BLK,),
        in_specs=[_row_spec, _row_spec, _row_spec, _deg_spec, _deg_spec,
                  _b_spec, _w_spec],
        out_specs=_row_spec, out_shape=_out_sds)(sa, sb, gp, da, db, b, w)


def _tc_final(sa, sb, gp, da, db, b):
    return pl.pallas_call(
        _tc_final_body, grid=(NP // BLK,),
        in_specs=[_row_spec, _row_spec, _row_spec, _deg_spec, _deg_spec,
                  _b_spec],
        out_specs=_row_spec, out_shape=_out_sds)(sa, sb, gp, da, db, b)


def kernel(x, edge_index, W0, b0, W1, b1, W2, b2):
    pad = jnp.full((EP - E,), N, jnp.int32)
    src = jnp.concatenate([edge_index[0], pad]).reshape(NW * CPW, CH)
    dst = jnp.concatenate([edge_index[1], pad]).reshape(NW * CPW, CH)
    x_pad = jnp.zeros((NP, D), f32).at[:N].set(x.astype(f32))
    onesD = jnp.ones((CH, D), f32)
    zerosD = jnp.zeros((RPT, D), f32)
    b0r, b1r, b2r = (b.reshape(1, D) for b in (b0, b1, b2))

    deg2 = _deg_kernel(dst, onesD, zerosD)
    da, db = deg2[:NP], deg2[NP:]

    g = _tc_first(x_pad, W0, da, db)
    s2 = _agg_kernel(g, src, dst, zerosD)
    g = _tc_mid(s2[:NP], s2[NP:], g, da, db, b0r, W1)
    s2 = _agg_kernel(g, src, dst, zerosD)
    g = _tc_mid(s2[:NP], s2[NP:], g, da, db, b1r, W2)
    s2 = _agg_kernel(g, src, dst, zerosD)
    out = _tc_final(s2[:NP], s2[NP:], g, da, db, b2r)
    return out[:N]


# speedup vs baseline: 23.5528x; 3.0376x over previous
import functools

import jax
import jax.numpy as jnp
from jax import lax
from jax.experimental import pallas as pl
from jax.experimental.pallas import tpu as pltpu
from jax.experimental.pallas import tpu_sc as plsc

N = 10000
NP = 10240
D = 128
E = 320000
NW = 32
CPW = 80
CH = 128
EP = NW * CPW * CH
RPT = NP // 16
IBLK = 16
BLK = 1280

f32 = jnp.float32

_mesh = plsc.VectorSubcoreMesh(core_axis_name="c", subcore_axis_name="s")


@functools.partial(
    pl.kernel,
    mesh=_mesh,
    out_type=jax.ShapeDtypeStruct((2 * NP, D), f32),
    scratch_types=[
        pltpu.VMEM_SHARED((NP, D), f32),
        pltpu.VMEM((CPW, CH), jnp.int32),
        pltpu.VMEM((CH, D), f32),
    ],
)
def _deg_kernel(dst_hbm, ones_hbm, zeros_hbm, out_hbm, acc, idx, ones_v):
    c = lax.axis_index("c")
    s = lax.axis_index("s")
    w = c * 16 + s
    pltpu.sync_copy(zeros_hbm, acc.at[pl.ds(s * RPT, RPT)])
    pltpu.sync_copy(ones_hbm, ones_v)
    pltpu.sync_copy(dst_hbm.at[pl.ds(w * CPW, CPW)], idx)
    plsc.subcore_barrier()

    def body(j, carry):
        pltpu.sync_copy(ones_v, acc.at[idx.at[j]], add=True)
        return carry

    lax.fori_loop(0, CPW, body, 0)
    plsc.subcore_barrier()
    pltpu.sync_copy(acc.at[pl.ds(s * RPT, RPT)],
                    out_hbm.at[pl.ds(c * NP + s * RPT, RPT)])


@functools.partial(
    pl.kernel,
    mesh=_mesh,
    out_type=jax.ShapeDtypeStruct((2 * NP, D), f32),
    scratch_types=[
        pltpu.VMEM_SHARED((NP, D), f32),
        pltpu.VMEM((3, IBLK, CH), jnp.int32),
        pltpu.VMEM((3, IBLK, CH), jnp.int32),
        pltpu.VMEM((CH, D), f32),
        pltpu.VMEM((CH, D), f32),
        pltpu.SemaphoreType.DMA,
        pltpu.SemaphoreType.DMA,
        pltpu.SemaphoreType.DMA,
    ],
)
def _agg_kernel(g_hbm, src_hbm, dst_hbm, zeros_hbm, out_hbm,
                acc, idx_s, idx_d, rows0, rows1, sem0, sem1, semi):
    c = lax.axis_index("c")
    s = lax.axis_index("s")
    w = c * 16 + s
    base = w * CPW
    pltpu.sync_copy(zeros_hbm, acc.at[pl.ds(s * RPT, RPT)])

    def iload(b, slot):
        pltpu.make_async_copy(
            src_hbm.at[pl.ds(base + b * IBLK, IBLK)], idx_s.at[slot], semi).start()
        pltpu.make_async_copy(
            dst_hbm.at[pl.ds(base + b * IBLK, IBLK)], idx_d.at[slot], semi).start()

    def iwait(b, slot):
        pltpu.make_async_copy(
            src_hbm.at[pl.ds(base + b * IBLK, IBLK)], idx_s.at[slot], semi).wait()
        pltpu.make_async_copy(
            dst_hbm.at[pl.ds(base + b * IBLK, IBLK)], idx_d.at[slot], semi).wait()

    def gather(j, buf, sem):
        return pltpu.make_async_copy(
            g_hbm.at[idx_s.at[(j // IBLK) % 3, j % IBLK]], buf, sem)

    def scat(j, buf):
        pltpu.sync_copy(buf, acc.at[idx_d.at[(j // IBLK) % 3, j % IBLK]],
                        add=True)

    iload(0, 0)
    iwait(0, 0)
    iload(1, 1)
    plsc.subcore_barrier()

    gather(0, rows0, sem0).start()

    def body(i, carry):
        j = 2 * i
        gather(j + 1, rows1, sem1).start()
        gather(j, rows0, sem0).wait()
        scat(j, rows0)

        @pl.when(i < CPW // 2 - 1)
        def _():
            @pl.when((j + 2) % IBLK == 0)
            def _():
                b2 = (j + 2) // IBLK
                iwait(b2, b2 % 3)

                @pl.when(b2 + 1 < CPW // IBLK)
                def _():
                    iload(b2 + 1, (b2 + 1) % 3)

            gather(j + 2, rows0, sem0).start()

        gather(j + 1, rows1, sem1).wait()
        scat(j + 1, rows1)
        return carry

    lax.fori_loop(0, CPW // 2, body, 0)
    plsc.subcore_barrier()
    pltpu.sync_copy(acc.at[pl.ds(s * RPT, RPT)],
                    out_hbm.at[pl.ds(c * NP + s * RPT, RPT)])


def _dinv_block(i, da_ref, db_ref):
    row = i * BLK + lax.broadcasted_iota(jnp.int32, (BLK, 1), 0)
    deg = da_ref[:, 0:1] + db_ref[:, 0:1] + 1.0
    return jnp.where(row < N, lax.rsqrt(deg), 0.0)


def _tc_first_body(x_ref, w_ref, da_ref, db_ref, g_ref):
    dinv = _dinv_block(pl.program_id(0), da_ref, db_ref)
    g_ref[...] = dinv * jnp.dot(x_ref[...], w_ref[...],
                                preferred_element_type=f32)


def _tc_mid_body(sa_ref, sb_ref, gp_ref, da_ref, db_ref, b_ref, w_ref, g_ref):
    dinv = _dinv_block(pl.program_id(0), da_ref, db_ref)
    h = jnp.maximum(
        dinv * (sa_ref[...] + sb_ref[...] + gp_ref[...]) + b_ref[...], 0.0)
    g_ref[...] = dinv * jnp.dot(h, w_ref[...], preferred_element_type=f32)


def _tc_final_body(sa_ref, sb_ref, gp_ref, da_ref, db_ref, b_ref, o_ref):
    dinv = _dinv_block(pl.program_id(0), da_ref, db_ref)
    o_ref[...] = dinv * (sa_ref[...] + sb_ref[...] + gp_ref[...]) + b_ref[...]


_row_spec = pl.BlockSpec((BLK, D), lambda i: (i, 0))
_deg_spec = _row_spec
_w_spec = pl.BlockSpec((D, D), lambda i: (0, 0))
_b_spec = pl.BlockSpec((1, D), lambda i: (0, 0))
_out_sds = jax.ShapeDtypeStruct((NP, D), f32)


def _tc_first(x, w, da, db):
    return pl.pallas_call(
        _tc_first_body, grid=(NP // BLK,),
        in_specs=[_row_spec, _w_spec, _deg_spec, _deg_spec],
        out_specs=_row_spec, out_shape=_out_sds)(x, w, da, db)


def _tc_mid(sa, sb, gp, da, db, b, w):
    return pl.pallas_call(
        _tc_mid_body, grid=(NP // BLK,),
        in_specs=[_row_spec, _row_spec, _row_spec, _deg_spec, _deg_spec,
                  _b_spec, _w_spec],
        out_specs=_row_spec, out_shape=_out_sds)(sa, sb, gp, da, db, b, w)


def _tc_final(sa, sb, gp, da, db, b):
    return pl.pallas_call(
        _tc_final_body, grid=(NP // BLK,),
        in_specs=[_row_spec, _row_spec, _row_spec, _deg_spec, _deg_spec,
                  _b_spec],
        out_specs=_row_spec, out_shape=_out_sds)(sa, sb, gp, da, db, b)


def kernel(x, edge_index, W0, b0, W1, b1, W2, b2):
    pad = N + (jnp.arange(EP - E, dtype=jnp.int32) % (NP - N))
    src = jnp.concatenate([edge_index[0], pad]).reshape(NW * CPW, CH)
    dst = jnp.concatenate([edge_index[1], pad]).reshape(NW * CPW, CH)
    x_pad = jnp.zeros((NP, D), f32).at[:N].set(x.astype(f32))
    onesD = jnp.ones((CH, D), f32)
    zerosD = jnp.zeros((RPT, D), f32)
    b0r, b1r, b2r = (b.reshape(1, D) for b in (b0, b1, b2))

    deg2 = _deg_kernel(dst, onesD, zerosD)
    da, db = deg2[:NP], deg2[NP:]

    g = _tc_first(x_pad, W0, da, db)
    s2 = _agg_kernel(g, src, dst, zerosD)
    g = _tc_mid(s2[:NP], s2[NP:], g, da, db, b0r, W1)
    s2 = _agg_kernel(g, src, dst, zerosD)
    g = _tc_mid(s2[:NP], s2[NP:], g, da, db, b1r, W2)
    s2 = _agg_kernel(g, src, dst, zerosD)
    out = _tc_final(s2[:NP], s2[NP:], g, da, db, b2r)
    return out[:N]


# speedup vs baseline: 24.7223x; 1.0497x over previous
import functools

import jax
import jax.numpy as jnp
from jax import lax
from jax.experimental import pallas as pl
from jax.experimental.pallas import tpu as pltpu
from jax.experimental.pallas import tpu_sc as plsc

N = 10000
NP = 10240
D = 128
E = 320000
NW = 32
CPW = 80
CH = 128
EP = NW * CPW * CH
RPT = NP // 16
IBLK = 16
BLK = 1280

f32 = jnp.float32

_mesh = plsc.VectorSubcoreMesh(core_axis_name="c", subcore_axis_name="s")


@functools.partial(
    pl.kernel,
    mesh=_mesh,
    out_type=jax.ShapeDtypeStruct((2 * NP, D), f32),
    scratch_types=[
        pltpu.VMEM_SHARED((NP, D), f32),
        pltpu.VMEM((CPW, CH), jnp.int32),
        pltpu.VMEM((CH, D), f32),
    ],
)
def _deg_kernel(dst_hbm, ones_hbm, zeros_hbm, out_hbm, acc, idx, ones_v):
    c = lax.axis_index("c")
    s = lax.axis_index("s")
    w = c * 16 + s
    pltpu.sync_copy(zeros_hbm, acc.at[pl.ds(s * RPT, RPT)])
    pltpu.sync_copy(ones_hbm, ones_v)
    pltpu.sync_copy(dst_hbm.at[pl.ds(w * CPW, CPW)], idx)
    plsc.subcore_barrier()

    def body(j, carry):
        pltpu.sync_copy(ones_v, acc.at[idx.at[j]], add=True)
        return carry

    lax.fori_loop(0, CPW, body, 0)
    plsc.subcore_barrier()
    pltpu.sync_copy(acc.at[pl.ds(s * RPT, RPT)],
                    out_hbm.at[pl.ds(c * NP + s * RPT, RPT)])


@functools.partial(
    pl.kernel,
    mesh=_mesh,
    out_type=jax.ShapeDtypeStruct((2 * NP, D), f32),
    scratch_types=[
        pltpu.VMEM_SHARED((NP, D), f32),
        pltpu.VMEM((3, IBLK, CH), jnp.int32),
        pltpu.VMEM((3, IBLK, CH), jnp.int32),
        pltpu.VMEM((CH, D), f32),
        pltpu.VMEM((CH, D), f32),
        pltpu.SemaphoreType.DMA,
        pltpu.SemaphoreType.DMA,
        pltpu.SemaphoreType.DMA,
    ],
)
def _agg_kernel(g_hbm, src_hbm, dst_hbm, zeros_hbm, out_hbm,
                acc, idx_s, idx_d, rows0, rows1, sem0, sem1, semi):
    c = lax.axis_index("c")
    s = lax.axis_index("s")
    w = c * 16 + s
    base = w * CPW
    pltpu.sync_copy(zeros_hbm, acc.at[pl.ds(s * RPT, RPT)])

    def iload(b, slot):
        pltpu.make_async_copy(
            src_hbm.at[pl.ds(base + b * IBLK, IBLK)], idx_s.at[slot], semi).start()
        pltpu.make_async_copy(
            dst_hbm.at[pl.ds(base + b * IBLK, IBLK)], idx_d.at[slot], semi).start()

    def iwait(b, slot):
        pltpu.make_async_copy(
            src_hbm.at[pl.ds(base + b * IBLK, IBLK)], idx_s.at[slot], semi).wait()
        pltpu.make_async_copy(
            dst_hbm.at[pl.ds(base + b * IBLK, IBLK)], idx_d.at[slot], semi).wait()

    def gather(j, buf, sem):
        return pltpu.make_async_copy(
            g_hbm.at[idx_s.at[(j // IBLK) % 3, j % IBLK]], buf, sem)

    def scat(j, buf):
        pltpu.sync_copy(buf, acc.at[idx_d.at[(j // IBLK) % 3, j % IBLK]],
                        add=True)

    iload(0, 0)
    iwait(0, 0)
    iload(1, 1)
    plsc.subcore_barrier()

    gather(0, rows0, sem0).start()

    def body(i, carry):
        j = 2 * i
        gather(j + 1, rows1, sem1).start()
        gather(j, rows0, sem0).wait()
        scat(j, rows0)

        @pl.when(i < CPW // 2 - 1)
        def _():
            @pl.when((j + 2) % IBLK == 0)
            def _():
                b2 = (j + 2) // IBLK
                iwait(b2, b2 % 3)

                @pl.when(b2 + 1 < CPW // IBLK)
                def _():
                    iload(b2 + 1, (b2 + 1) % 3)

            gather(j + 2, rows0, sem0).start()

        gather(j + 1, rows1, sem1).wait()
        scat(j + 1, rows1)
        return carry

    lax.fori_loop(0, CPW // 2, body, 0)
    plsc.subcore_barrier()
    pltpu.sync_copy(acc.at[pl.ds(s * RPT, RPT)],
                    out_hbm.at[pl.ds(c * NP + s * RPT, RPT)])


def _dinv_block(i, da_ref, db_ref):
    row = i * BLK + lax.broadcasted_iota(jnp.int32, (BLK, 1), 0)
    deg = da_ref[:, 0:1] + db_ref[:, 0:1] + 1.0
    return jnp.where(row < N, lax.rsqrt(deg), 0.0)


def _tc_first_body(x_ref, w_ref, da_ref, db_ref, g_ref):
    dinv = _dinv_block(pl.program_id(0), da_ref, db_ref)
    g_ref[...] = dinv * jnp.dot(x_ref[...], w_ref[...],
                                preferred_element_type=f32)


def _tc_mid_body(sa_ref, sb_ref, gp_ref, da_ref, db_ref, b_ref, w_ref, g_ref):
    dinv = _dinv_block(pl.program_id(0), da_ref, db_ref)
    h = jnp.maximum(
        dinv * (sa_ref[...] + sb_ref[...] + gp_ref[...]) + b_ref[...], 0.0)
    g_ref[...] = dinv * jnp.dot(h, w_ref[...], preferred_element_type=f32)


def _tc_final_body(sa_ref, sb_ref, gp_ref, da_ref, db_ref, b_ref, o_ref):
    dinv = _dinv_block(pl.program_id(0), da_ref, db_ref)
    o_ref[...] = dinv * (sa_ref[...] + sb_ref[...] + gp_ref[...]) + b_ref[...]


_row_spec = pl.BlockSpec((BLK, D), lambda i: (i, 0))
_lo_spec = pl.BlockSpec((BLK, D), lambda i: (i, 0))
_hi_spec = pl.BlockSpec((BLK, D), lambda i: (i + NP // BLK, 0))
_w_spec = pl.BlockSpec((D, D), lambda i: (0, 0))
_b_spec = pl.BlockSpec((1, D), lambda i: (0, 0))
_out_sds = jax.ShapeDtypeStruct((NP, D), f32)


def _tc_first(x, w, deg2):
    return pl.pallas_call(
        _tc_first_body, grid=(NP // BLK,),
        in_specs=[_row_spec, _w_spec, _lo_spec, _hi_spec],
        out_specs=_row_spec, out_shape=_out_sds)(x, w, deg2, deg2)


def _tc_mid(s2, gp, deg2, b, w):
    return pl.pallas_call(
        _tc_mid_body, grid=(NP // BLK,),
        in_specs=[_lo_spec, _hi_spec, _row_spec, _lo_spec, _hi_spec,
                  _b_spec, _w_spec],
        out_specs=_row_spec, out_shape=_out_sds)(s2, s2, gp, deg2, deg2, b, w)


def _tc_final(s2, gp, deg2, b):
    return pl.pallas_call(
        _tc_final_body, grid=(NP // BLK,),
        in_specs=[_lo_spec, _hi_spec, _row_spec, _lo_spec, _hi_spec,
                  _b_spec],
        out_specs=_row_spec, out_shape=_out_sds)(s2, s2, gp, deg2, deg2, b)


def kernel(x, edge_index, W0, b0, W1, b1, W2, b2):
    pad = N + (jnp.arange(EP - E, dtype=jnp.int32) % (NP - N))
    src = jnp.concatenate([edge_index[0], pad]).reshape(NW * CPW, CH)
    dst = jnp.concatenate([edge_index[1], pad]).reshape(NW * CPW, CH)
    x_pad = jnp.zeros((NP, D), f32).at[:N].set(x.astype(f32))
    onesD = jnp.ones((CH, D), f32)
    zerosD = jnp.zeros((RPT, D), f32)
    b0r, b1r, b2r = (b.reshape(1, D) for b in (b0, b1, b2))

    deg2 = _deg_kernel(dst, onesD, zerosD)

    g = _tc_first(x_pad, W0, deg2)
    s2 = _agg_kernel(g, src, dst, zerosD)
    g = _tc_mid(s2, g, deg2, b0r, W1)
    s2 = _agg_kernel(g, src, dst, zerosD)
    g = _tc_mid(s2, g, deg2, b1r, W2)
    s2 = _agg_kernel(g, src, dst, zerosD)
    out = _tc_final(s2, g, deg2, b2r)
    return out[:N]


# speedup vs baseline: 25.4694x; 1.0302x over previous
import functools

import jax
import jax.numpy as jnp
from jax import lax
from jax.experimental import pallas as pl
from jax.experimental.pallas import tpu as pltpu
from jax.experimental.pallas import tpu_sc as plsc

N = 10000
NP = 10240
D = 128
E = 320000
NW = 32
CPW = 160
CH = 64
EP = NW * CPW * CH
RPT = NP // 16
BLK = 1280
NB = 4
IBLK = 16
NBLK = CPW // IBLK
QD = 6

f32 = jnp.float32

_mesh = plsc.VectorSubcoreMesh(core_axis_name="c", subcore_axis_name="s")


def _zero_acc(zeros_hbm, tile, acc, s):
    pltpu.sync_copy(zeros_hbm, tile)
    for k in range(RPT // CH):
        pltpu.sync_copy(tile, acc.at[pl.ds(s * RPT + k * CH, CH)])


@functools.partial(
    pl.kernel,
    mesh=_mesh,
    out_type=jax.ShapeDtypeStruct((2 * NP, D), f32),
    scratch_types=[
        pltpu.VMEM_SHARED((NP, D), f32),
        pltpu.VMEM((CPW, CH), jnp.int32),
        pltpu.VMEM((CH, D), f32),
        pltpu.SemaphoreType.DMA,
    ],
)
def _deg_kernel(dst_hbm, ones_hbm, zeros_hbm, out_hbm, acc, idx, ones_v, sem):
    c = lax.axis_index("c")
    s = lax.axis_index("s")
    w = c * 16 + s
    pltpu.sync_copy(dst_hbm.at[pl.ds(w * CPW, CPW)], idx)
    _zero_acc(zeros_hbm, ones_v, acc, s)
    pltpu.sync_copy(ones_hbm, ones_v)
    plsc.subcore_barrier()

    def scat(j):
        return pltpu.make_async_copy(ones_v, acc.at[idx.at[j]], sem)

    for b in range(QD):
        pltpu.async_copy(ones_v, acc.at[idx.at[b]], sem, add=True)

    def body(i, carry):
        scat(i).wait()
        pltpu.async_copy(ones_v, acc.at[idx.at[i + QD]], sem, add=True)
        return carry

    lax.fori_loop(0, CPW - QD, body, 0)
    for b in range(QD):
        scat(CPW - QD + b).wait()

    plsc.subcore_barrier()
    pltpu.sync_copy(acc.at[pl.ds(s * RPT, RPT)],
                    out_hbm.at[pl.ds(c * NP + s * RPT, RPT)])


@functools.partial(
    pl.kernel,
    mesh=_mesh,
    out_type=jax.ShapeDtypeStruct((2 * NP, D), f32),
    scratch_types=[
        pltpu.VMEM_SHARED((NP, D), f32),
        pltpu.VMEM((3, IBLK, CH), jnp.int32),
        pltpu.VMEM((3, IBLK, CH), jnp.int32),
        pltpu.VMEM((NB, CH, D), f32),
        pltpu.SemaphoreType.DMA,
        pltpu.SemaphoreType.DMA,
        pltpu.SemaphoreType.DMA,
        pltpu.SemaphoreType.DMA,
        pltpu.SemaphoreType.DMA,
        pltpu.SemaphoreType.DMA,
        pltpu.SemaphoreType.DMA,
        pltpu.SemaphoreType.DMA,
        pltpu.SemaphoreType.DMA,
    ],
)
def _agg_kernel(g_hbm, src_hbm, dst_hbm, zeros_hbm, out_hbm,
                acc, idx_s, idx_d, rows,
                sg0, sg1, sg2, sg3, ss0, ss1, ss2, ss3, semi):
    sg = [sg0, sg1, sg2, sg3]
    ss = [ss0, ss1, ss2, ss3]
    c = lax.axis_index("c")
    s = lax.axis_index("s")
    w = c * 16 + s
    base = w * CPW
    _zero_acc(zeros_hbm, rows.at[0], acc, s)

    def iload(b, slot):
        pltpu.make_async_copy(
            src_hbm.at[pl.ds(base + b * IBLK, IBLK)], idx_s.at[slot], semi).start()
        pltpu.make_async_copy(
            dst_hbm.at[pl.ds(base + b * IBLK, IBLK)], idx_d.at[slot], semi).start()

    def iwait(b, slot):
        pltpu.make_async_copy(
            src_hbm.at[pl.ds(base + b * IBLK, IBLK)], idx_s.at[slot], semi).wait()
        pltpu.make_async_copy(
            dst_hbm.at[pl.ds(base + b * IBLK, IBLK)], idx_d.at[slot], semi).wait()

    def sidx(j):
        return idx_s.at[(j // IBLK) % 3, j % IBLK]

    def didx(j):
        return idx_d.at[(j // IBLK) % 3, j % IBLK]

    def gath(j, b):
        return pltpu.make_async_copy(g_hbm.at[sidx(j)], rows.at[b], sg[b])

    def scat(j, b):
        return pltpu.make_async_copy(rows.at[b], acc.at[didx(j)], ss[b])

    iload(0, 0)
    iwait(0, 0)
    iload(1, 1)
    plsc.subcore_barrier()

    for b in range(NB - 1):
        pltpu.async_copy(g_hbm.at[sidx(b)], rows.at[b], sg[b])

    def body(i, carry):
        j0 = i * NB
        for b in range(NB):
            j = j0 + b
            gath(j, b).wait()
            pltpu.async_copy(rows.at[b], acc.at[didx(j)], ss[b], add=True)
            bm1 = (b - 1) % NB
            if b == 0:
                @pl.when(j0 >= 1)
                def _():
                    scat(j0 - 1, bm1).wait()
            else:
                scat(j - 1, bm1).wait()
            b3 = (b + 3) % NB

            @pl.when(j + 3 < CPW)
            def _():
                @pl.when(lax.rem(j + 3, IBLK) == 0)
                def _():
                    b2 = (j + 3) // IBLK
                    iwait(b2, b2 % 3)

                    @pl.when(b2 + 1 < NBLK)
                    def _():
                        iload(b2 + 1, (b2 + 1) % 3)

                pltpu.async_copy(g_hbm.at[sidx(j + 3)], rows.at[b3], sg[b3])
        return carry

    lax.fori_loop(0, CPW // NB, body, 0)
    scat(CPW - 1, (CPW - 1) % NB).wait()
    plsc.subcore_barrier()
    pltpu.sync_copy(acc.at[pl.ds(s * RPT, RPT)],
                    out_hbm.at[pl.ds(c * NP + s * RPT, RPT)])


def _dinv_block(i, da_ref, db_ref):
    row = i * BLK + lax.broadcasted_iota(jnp.int32, (BLK, 1), 0)
    deg = da_ref[:, 0:1] + db_ref[:, 0:1] + 1.0
    return jnp.where(row < N, lax.rsqrt(deg), 0.0)


def _tc_first_body(x_ref, w_ref, da_ref, db_ref, g_ref):
    dinv = _dinv_block(pl.program_id(0), da_ref, db_ref)
    g_ref[...] = dinv * jnp.dot(x_ref[...], w_ref[...],
                                preferred_element_type=f32)


def _tc_mid_body(sa_ref, sb_ref, gp_ref, da_ref, db_ref, b_ref, w_ref, g_ref):
    dinv = _dinv_block(pl.program_id(0), da_ref, db_ref)
    h = jnp.maximum(
        dinv * (sa_ref[...] + sb_ref[...] + gp_ref[...]) + b_ref[...], 0.0)
    g_ref[...] = dinv * jnp.dot(h, w_ref[...], preferred_element_type=f32)


def _tc_final_body(sa_ref, sb_ref, gp_ref, da_ref, db_ref, b_ref, o_ref):
    dinv = _dinv_block(pl.program_id(0), da_ref, db_ref)
    o_ref[...] = dinv * (sa_ref[...] + sb_ref[...] + gp_ref[...]) + b_ref[...]


_row_spec = pl.BlockSpec((BLK, D), lambda i: (i, 0))
_lo_spec = pl.BlockSpec((BLK, D), lambda i: (i, 0))
_hi_spec = pl.BlockSpec((BLK, D), lambda i: (i + NP // BLK, 0))
_w_spec = pl.BlockSpec((D, D), lambda i: (0, 0))
_b_spec = pl.BlockSpec((1, D), lambda i: (0, 0))
_out_sds = jax.ShapeDtypeStruct((NP, D), f32)


def _tc_first(x, w, deg2):
    return pl.pallas_call(
        _tc_first_body, grid=(NP // BLK,),
        in_specs=[_row_spec, _w_spec, _lo_spec, _hi_spec],
        out_specs=_row_spec, out_shape=_out_sds)(x, w, deg2, deg2)


def _tc_mid(s2, gp, deg2, b, w):
    return pl.pallas_call(
        _tc_mid_body, grid=(NP // BLK,),
        in_specs=[_lo_spec, _hi_spec, _row_spec, _lo_spec, _hi_spec,
                  _b_spec, _w_spec],
        out_specs=_row_spec, out_shape=_out_sds)(s2, s2, gp, deg2, deg2, b, w)


def _tc_final(s2, gp, deg2, b):
    return pl.pallas_call(
        _tc_final_body, grid=(NP // BLK,),
        in_specs=[_lo_spec, _hi_spec, _row_spec, _lo_spec, _hi_spec,
                  _b_spec],
        out_specs=_row_spec, out_shape=_out_sds)(s2, s2, gp, deg2, deg2, b)


def kernel(x, edge_index, W0, b0, W1, b1, W2, b2):
    pad = N + (jnp.arange(EP - E, dtype=jnp.int32) % (NP - N))
    src = jnp.concatenate([edge_index[0], pad]).reshape(NW * CPW, CH)
    dst = jnp.concatenate([edge_index[1], pad]).reshape(NW * CPW, CH)
    x_pad = jnp.zeros((NP, D), f32).at[:N].set(x.astype(f32))
    onesC = jnp.ones((CH, D), f32)
    zerosC = jnp.zeros((CH, D), f32)
    b0r, b1r, b2r = (b.reshape(1, D) for b in (b0, b1, b2))

    deg2 = _deg_kernel(dst, onesC, zerosC)

    g = _tc_first(x_pad, W0, deg2)
    s2 = _agg_kernel(g, src, dst, zerosC)
    g = _tc_mid(s2, g, deg2, b0r, W1)
    s2 = _agg_kernel(g, src, dst, zerosC)
    g = _tc_mid(s2, g, deg2, b1r, W2)
    s2 = _agg_kernel(g, src, dst, zerosC)
    out = _tc_final(s2, g, deg2, b2r)
    return out[:N]


# speedup vs baseline: 25.7039x; 1.0092x over previous
import functools

import jax
import jax.numpy as jnp
from jax import lax
from jax.experimental import pallas as pl
from jax.experimental.pallas import tpu as pltpu
from jax.experimental.pallas import tpu_sc as plsc

N = 10000
NP = 10240
D = 128
E = 320000
NW = 32
CPW = 160
CH = 64
EP = NW * CPW * CH
RPT = NP // 16
BLK = 1280
NB = 4
IBLK = 16
NBLK = CPW // IBLK
QD = 6

f32 = jnp.float32

_mesh = plsc.VectorSubcoreMesh(core_axis_name="c", subcore_axis_name="s")


def _zero_acc(zeros_hbm, tile, acc, s):
    pltpu.sync_copy(zeros_hbm, tile)
    for k in range(RPT // CH):
        pltpu.sync_copy(tile, acc.at[pl.ds(s * RPT + k * CH, CH)])


@functools.partial(
    pl.kernel,
    mesh=_mesh,
    out_type=jax.ShapeDtypeStruct((2 * NP, D), f32),
    scratch_types=[
        pltpu.VMEM_SHARED((NP, D), f32),
        pltpu.VMEM((CPW, CH), jnp.int32),
        pltpu.VMEM((CH, D), f32),
        pltpu.SemaphoreType.DMA,
    ],
)
def _deg_kernel(dst_hbm, ones_hbm, zeros_hbm, out_hbm, acc, idx, ones_v, sem):
    c = lax.axis_index("c")
    s = lax.axis_index("s")
    w = c * 16 + s
    pltpu.sync_copy(dst_hbm.at[pl.ds(w * CPW, CPW)], idx)
    _zero_acc(zeros_hbm, ones_v, acc, s)
    pltpu.sync_copy(ones_hbm, ones_v)
    plsc.subcore_barrier()

    def scat(j):
        return pltpu.make_async_copy(ones_v, acc.at[idx.at[j]], sem)

    for b in range(QD):
        pltpu.async_copy(ones_v, acc.at[idx.at[b]], sem, add=True)

    def body(i, carry):
        scat(i).wait()
        pltpu.async_copy(ones_v, acc.at[idx.at[i + QD]], sem, add=True)
        return carry

    lax.fori_loop(0, CPW - QD, body, 0)
    for b in range(QD):
        scat(CPW - QD + b).wait()

    plsc.subcore_barrier()
    pltpu.sync_copy(acc.at[pl.ds(s * RPT, RPT)],
                    out_hbm.at[pl.ds(c * NP + s * RPT, RPT)])


@functools.partial(
    pl.kernel,
    mesh=_mesh,
    out_type=jax.ShapeDtypeStruct((2 * NP, D), f32),
    scratch_types=[
        pltpu.VMEM_SHARED((NP, D), f32),
        pltpu.VMEM((3, IBLK, CH), jnp.int32),
        pltpu.VMEM((3, IBLK, CH), jnp.int32),
        pltpu.VMEM((NB, CH, D), f32),
        pltpu.SemaphoreType.DMA,
        pltpu.SemaphoreType.DMA,
        pltpu.SemaphoreType.DMA,
        pltpu.SemaphoreType.DMA,
        pltpu.SemaphoreType.DMA,
        pltpu.SemaphoreType.DMA,
        pltpu.SemaphoreType.DMA,
        pltpu.SemaphoreType.DMA,
        pltpu.SemaphoreType.DMA,
    ],
)
def _agg_kernel(g_hbm, src_hbm, dst_hbm, zeros_hbm, out_hbm,
                acc, idx_s, idx_d, rows,
                sg0, sg1, sg2, sg3, ss0, ss1, ss2, ss3, semi):
    sg = [sg0, sg1, sg2, sg3]
    ss = [ss0, ss1, ss2, ss3]
    c = lax.axis_index("c")
    s = lax.axis_index("s")
    w = c * 16 + s
    base = w * CPW
    _zero_acc(zeros_hbm, rows.at[0], acc, s)

    def iload(b, slot):
        pltpu.make_async_copy(
            src_hbm.at[pl.ds(base + b * IBLK, IBLK)], idx_s.at[slot], semi).start()
        pltpu.make_async_copy(
            dst_hbm.at[pl.ds(base + b * IBLK, IBLK)], idx_d.at[slot], semi).start()

    def iwait(b, slot):
        pltpu.make_async_copy(
            src_hbm.at[pl.ds(base + b * IBLK, IBLK)], idx_s.at[slot], semi).wait()
        pltpu.make_async_copy(
            dst_hbm.at[pl.ds(base + b * IBLK, IBLK)], idx_d.at[slot], semi).wait()

    def sidx(j):
        return idx_s.at[(j // IBLK) % 3, j % IBLK]

    def didx(j):
        return idx_d.at[(j // IBLK) % 3, j % IBLK]

    def gath(j, b):
        return pltpu.make_async_copy(g_hbm.at[sidx(j)], rows.at[b], sg[b])

    def scat(j, b):
        return pltpu.make_async_copy(rows.at[b], acc.at[didx(j)], ss[b])

    iload(0, 0)
    iwait(0, 0)
    iload(1, 1)
    plsc.subcore_barrier()

    for b in range(NB - 1):
        pltpu.async_copy(g_hbm.at[sidx(b)], rows.at[b], sg[b])

    def body(i, carry):
        j0 = i * NB
        for b in range(NB):
            j = j0 + b
            gath(j, b).wait()
            pltpu.async_copy(rows.at[b], acc.at[didx(j)], ss[b], add=True)
            bm1 = (b - 1) % NB
            if b == 0:
                @pl.when(j0 >= 1)
                def _():
                    scat(j0 - 1, bm1).wait()
            else:
                scat(j - 1, bm1).wait()
            b3 = (b + 3) % NB

            @pl.when(j + 3 < CPW)
            def _():
                @pl.when(lax.rem(j + 3, IBLK) == 0)
                def _():
                    b2 = (j + 3) // IBLK
                    iwait(b2, b2 % 3)

                    @pl.when(b2 + 1 < NBLK)
                    def _():
                        iload(b2 + 1, (b2 + 1) % 3)

                pltpu.async_copy(g_hbm.at[sidx(j + 3)], rows.at[b3], sg[b3])
        return carry

    lax.fori_loop(0, CPW // NB, body, 0)
    scat(CPW - 1, (CPW - 1) % NB).wait()
    plsc.subcore_barrier()
    pltpu.sync_copy(acc.at[pl.ds(s * RPT, RPT)],
                    out_hbm.at[pl.ds(c * NP + s * RPT, RPT)])


def _dinv_block(i, da_ref, db_ref):
    row = i * BLK + lax.broadcasted_iota(jnp.int32, (BLK, 1), 0)
    deg = da_ref[:, 0:1] + db_ref[:, 0:1] + 1.0
    return jnp.where(row < N, lax.rsqrt(deg), 0.0)


def _tc_first_body(x_ref, w_ref, da_ref, db_ref, g_ref, dv_ref):
    dinv = _dinv_block(pl.program_id(0), da_ref, db_ref)
    g_ref[...] = dinv * jnp.dot(x_ref[...], w_ref[...],
                                preferred_element_type=f32)
    dv_ref[...] = jnp.broadcast_to(dinv, (BLK, D))


def _tc_mid_body(sa_ref, sb_ref, gp_ref, dv_ref, b_ref, w_ref, g_ref):
    dinv = dv_ref[:, 0:1]
    h = jnp.maximum(
        dinv * (sa_ref[...] + sb_ref[...] + gp_ref[...]) + b_ref[...], 0.0)
    g_ref[...] = dinv * jnp.dot(h, w_ref[...], preferred_element_type=f32)


def _tc_final_body(sa_ref, sb_ref, gp_ref, dv_ref, b_ref, o_ref):
    dinv = dv_ref[:, 0:1]
    o_ref[...] = dinv * (sa_ref[...] + sb_ref[...] + gp_ref[...]) + b_ref[...]


_row_spec = pl.BlockSpec((BLK, D), lambda i: (i, 0))
_lo_spec = pl.BlockSpec((BLK, D), lambda i: (i, 0))
_hi_spec = pl.BlockSpec((BLK, D), lambda i: (i + NP // BLK, 0))
_w_spec = pl.BlockSpec((D, D), lambda i: (0, 0))
_b_spec = pl.BlockSpec((1, D), lambda i: (0, 0))
_out_sds = jax.ShapeDtypeStruct((NP, D), f32)


def _tc_first(x, w, deg2):
    return pl.pallas_call(
        _tc_first_body, grid=(NP // BLK,),
        in_specs=[_row_spec, _w_spec, _lo_spec, _hi_spec],
        out_specs=[_row_spec, _row_spec],
        out_shape=[_out_sds, _out_sds])(x, w, deg2, deg2)


def _tc_mid(s2, gp, dv, b, w):
    return pl.pallas_call(
        _tc_mid_body, grid=(NP // BLK,),
        in_specs=[_lo_spec, _hi_spec, _row_spec, _row_spec,
                  _b_spec, _w_spec],
        out_specs=_row_spec, out_shape=_out_sds)(s2, s2, gp, dv, b, w)


def _tc_final(s2, gp, dv, b):
    return pl.pallas_call(
        _tc_final_body, grid=(NP // BLK,),
        in_specs=[_lo_spec, _hi_spec, _row_spec, _row_spec,
                  _b_spec],
        out_specs=_row_spec, out_shape=_out_sds)(s2, s2, gp, dv, b)


def kernel(x, edge_index, W0, b0, W1, b1, W2, b2):
    pad = N + (jnp.arange(EP - E, dtype=jnp.int32) % (NP - N))
    src = jnp.concatenate([edge_index[0], pad]).reshape(NW * CPW, CH)
    dst = jnp.concatenate([edge_index[1], pad]).reshape(NW * CPW, CH)
    x_pad = jnp.zeros((NP, D), f32).at[:N].set(x.astype(f32))
    onesC = jnp.ones((CH, D), f32)
    zerosC = jnp.zeros((CH, D), f32)
    b0r, b1r, b2r = (b.reshape(1, D) for b in (b0, b1, b2))

    deg2 = _deg_kernel(dst, onesC, zerosC)

    g, dv = _tc_first(x_pad, W0, deg2)
    s2 = _agg_kernel(g, src, dst, zerosC)
    g = _tc_mid(s2, g, dv, b0r, W1)
    s2 = _agg_kernel(g, src, dst, zerosC)
    g = _tc_mid(s2, g, dv, b1r, W2)
    s2 = _agg_kernel(g, src, dst, zerosC)
    out = _tc_final(s2, g, dv, b2r)
    return out[:N]
